# baseline (speedup 1.0000x reference)
import functools

import jax
import jax.numpy as jnp
from jax import lax
from jax.experimental import pallas as pl
from jax.experimental.pallas import tpu as pltpu
from jax.experimental.pallas import tpu_sc as plsc

NUM_NODES = 1000000
MEM_DIM = 32
B = 16384

NC = 2
NS = 16
NW = NC * NS
CHUNK = 128
ROWS_PER_W = B // NW
NCHUNK = ROWS_PER_W // CHUNK

NCT = NUM_NODES // 128
EDGE_COLS = NUM_NODES - NCT * 128
BASE_CT = NCT // NW
EXTRA_CT = NCT - BASE_CT * NW
MCAP = 768
NIDCH = B // 16
NSCAN = MCAP // 16

@functools.lru_cache(maxsize=None)
def _get_sc_kernels():
    mesh = plsc.VectorSubcoreMesh(core_axis_name="c", subcore_axis_name="s",
                                  num_cores=NC, num_subcores=NS)
    sc_params = pltpu.CompilerParams(use_tc_tiling_on_sc=False)

    @functools.partial(
        pl.kernel,
        mesh=mesh,
        out_type=(
            jax.ShapeDtypeStruct((B // CHUNK, CHUNK, MEM_DIM), jnp.float32),
            jax.ShapeDtypeStruct((B // CHUNK, CHUNK, MEM_DIM), jnp.float32),
        ),
        scratch_types=[
            pltpu.VMEM((NCHUNK, CHUNK), jnp.int32),
            pltpu.VMEM((NCHUNK, CHUNK), jnp.int32),
            pltpu.VMEM((NCHUNK, CHUNK, MEM_DIM), jnp.float32),
            pltpu.VMEM((NCHUNK, CHUNK, MEM_DIM), jnp.float32),
            pltpu.SemaphoreType.DMA,
        ],
        compiler_params=sc_params,
    )
    def sc_gather(table_hbm, src_hbm, dst_hbm, src_out, dst_out,
                  sidx_v, didx_v, srows_v, drows_v, sem):
        wid = lax.axis_index("s") * NC + lax.axis_index("c")
        base = wid * NCHUNK
        pltpu.sync_copy(src_hbm.at[pl.ds(base, NCHUNK)], sidx_v)
        pltpu.sync_copy(dst_hbm.at[pl.ds(base, NCHUNK)], didx_v)
        copies = []
        for c in range(NCHUNK):
            copies.append(
                pltpu.async_copy(table_hbm.at[sidx_v.at[c]], srows_v.at[c], sem))
            copies.append(
                pltpu.async_copy(table_hbm.at[didx_v.at[c]], drows_v.at[c], sem))
        for cp in copies:
            cp.wait()
        pltpu.sync_copy(srows_v, src_out.at[pl.ds(base, NCHUNK)])
        pltpu.sync_copy(drows_v, dst_out.at[pl.ds(base, NCHUNK)])

    cs_params = pltpu.CompilerParams(use_tc_tiling_on_sc=True,
                                     needs_layout_passes=False)
    iota16 = lambda: lax.iota(jnp.int32, 16)

    @functools.partial(
        pl.kernel,
        mesh=mesh,
        out_type=jax.ShapeDtypeStruct((MEM_DIM, NUM_NODES), jnp.float32),
        scratch_types=[
            pltpu.VMEM((B,), jnp.int32),
            pltpu.VMEM((MCAP,), jnp.int32),
            pltpu.VMEM((MCAP,), jnp.int32),
            pltpu.VMEM((MCAP, 128), jnp.float32),
            pltpu.VMEM((MEM_DIM, 128), jnp.float32),
            pltpu.SemaphoreType.DMA,
        ],
        compiler_params=cs_params,
    )
    def copy_scatter(mem_t, dst_hbm, upd_hbm, out_t,
                     ids_v, mid_v, mevt_v, upd_v, slab_v, sem):
        wid = lax.axis_index("s") * NC + lax.axis_index("c")
        extra = jnp.minimum(wid, EXTRA_CT)
        start_ct = wid * BASE_CT + extra
        nct = BASE_CT + jnp.where(wid < EXTRA_CT, 1, 0)
        lo = start_ct * 128
        hi = lo + nct * 128

        neg16 = jnp.full((16,), -1, jnp.int32)
        zero16 = jnp.zeros((16,), jnp.int32)
        for z in range(NSCAN):
            mid_v[pl.ds(z * 16, 16)] = neg16
            mevt_v[pl.ds(z * 16, 16)] = zero16

        pltpu.sync_copy(dst_hbm, ids_v)

        def ebody(i, cnt):
            vec = ids_v[pl.ds(i * 16, 16)]
            mask = (vec >= lo) & (vec < hi)
            mi = mask.astype(jnp.int32)
            pos = cnt + plsc.cumsum(mi) - 1
            plsc.store_scatter(mid_v, [pos], vec, mask=mask)
            plsc.store_scatter(mevt_v, [pos], i * 16 + iota16(), mask=mask)
            return cnt + jnp.sum(mi)

        cnt = lax.fori_loop(0, NIDCH, ebody, jnp.int32(0))

        gcopies = [
            pltpu.async_copy(upd_hbm.at[mevt_v.at[pl.ds(j * 128, 128)]],
                             upd_v.at[pl.ds(j * 128, 128)], sem)
            for j in range(MCAP // 128)
        ]
        for cp in gcopies:
            cp.wait()

        def cbody(ct, _):
            c0 = pl.multiple_of((start_ct + ct) * 128, 128)
            pltpu.sync_copy(mem_t.at[:, pl.ds(c0, 128)], slab_v)

            def sbody(j, _s):
                evec = jnp.full((16,), j * 16, jnp.int32) + iota16()
                vec = mid_v[pl.ds(j * 16, 16)]
                msk = (vec >= c0) & (vec < c0 + 128) & (evec < cnt)
                nhit = jnp.sum(msk.astype(jnp.int32))

                @pl.when(nhit > 0)
                def _():
                    col = vec - c0
                    for d in range(MEM_DIM):
                        dfull = jnp.full((16,), d, jnp.int32)
                        vals = plsc.load_gather(upd_v, [evec, dfull])
                        plsc.store_scatter(slab_v, [dfull, col], vals, mask=msk)
                return 0

            lax.fori_loop(0, NSCAN, sbody, 0)
            pltpu.sync_copy(slab_v, out_t.at[:, pl.ds(c0, 128)])
            return 0

        lax.fori_loop(0, nct, cbody, 0)

    return sc_gather, copy_scatter


_RBLK = 2048


def _tc_body(src_ref, dst_ref, edge_ref, dt_ref,
             w1s_ref, w1d_ref, w1e_ref, w1t_ref, b1_ref,
             w2_ref, b2_ref,
             wih_r_ref, wih_z_ref, wih_n_ref,
             whh_r_ref, whh_z_ref, whh_n_ref,
             bi_r_ref, bi_z_ref, bi_n_ref,
             bh_r_ref, bh_z_ref, bh_n_ref,
             wv_ref, bv_ref, wout_ref, bout_ref,
             we1a_ref, we1e_ref, be1_ref, we2_ref, be2_ref,
             wc1_ref, bc1_ref, wc2_ref, bc2_ref,
             upd_ref, probs_ref):
    src = src_ref[...]
    dst = dst_ref[...]
    edge = edge_ref[...]
    dt = dt_ref[...]

    def mm(a, w):
        return jnp.dot(a, w[...], preferred_element_type=jnp.float32)

    h = mm(src, w1s_ref) + mm(dst, w1d_ref) + mm(edge, w1e_ref) \
        + dt * w1t_ref[...] + b1_ref[...]
    h = jnp.maximum(h, 0.0)
    msg = mm(h, w2_ref) + b2_ref[...]

    r = jax.nn.sigmoid(mm(msg, wih_r_ref) + bi_r_ref[...]
                       + mm(dst, whh_r_ref) + bh_r_ref[...])
    z = jax.nn.sigmoid(mm(msg, wih_z_ref) + bi_z_ref[...]
                       + mm(dst, whh_z_ref) + bh_z_ref[...])
    n = jnp.tanh(mm(msg, wih_n_ref) + bi_n_ref[...]
                 + r * (mm(dst, whh_n_ref) + bh_n_ref[...]))
    updated = (1.0 - z) * n + z * dst
    upd_ref[...] = jnp.concatenate(
        [updated, jnp.zeros((updated.shape[0], 128 - MEM_DIM), jnp.float32)],
        axis=1)

    v = mm(dst, wv_ref) + bv_ref[...]
    attn_out = mm(v, wout_ref) + bout_ref[...]
    e = jnp.maximum(mm(attn_out, we1a_ref) + mm(edge, we1e_ref) + be1_ref[...], 0.0)
    e = mm(e, we2_ref) + be2_ref[...]

    c = jnp.maximum(mm(e, wc1_ref) + bc1_ref[...], 0.0)
    logits = mm(c, wc2_ref) + bc2_ref[...]
    probs_ref[...] = jax.nn.sigmoid(logits)


_TW = 2048
_TR = _TW * MEM_DIM // 128
_TGRID = -(-NUM_NODES // _TW)
_PACKED_ROWS = NUM_NODES * MEM_DIM // 128


def _to_linear_body(mem_t_ref, out_ref):
    t1 = mem_t_ref[...].T
    t3 = t1.reshape(_TR, 4, MEM_DIM)
    out_ref[...] = jnp.concatenate([t3[:, a, :] for a in range(4)], axis=1)


def _from_linear_body(lin_ref, out_ref):
    blk = lin_ref[...]
    parts = [blk[:, MEM_DIM * a:MEM_DIM * (a + 1)] for a in range(4)]
    st = jnp.stack(parts, axis=1)
    out_ref[...] = st.reshape(_TW, MEM_DIM).T


def _to_linear(mem_t):
    return pl.pallas_call(
        _to_linear_body,
        grid=(_TGRID,),
        in_specs=[pl.BlockSpec((MEM_DIM, _TW), lambda i: (0, i))],
        out_specs=pl.BlockSpec((_TR, 128), lambda i: (i, 0)),
        out_shape=jax.ShapeDtypeStruct((_PACKED_ROWS, 128), jnp.float32),
        name="table_to_linear",
    )(mem_t)


def _from_linear(lin):
    return pl.pallas_call(
        _from_linear_body,
        grid=(_TGRID,),
        in_specs=[pl.BlockSpec((_TR, 128), lambda i: (i, 0))],
        out_specs=pl.BlockSpec((MEM_DIM, _TW), lambda i: (0, i)),
        out_shape=jax.ShapeDtypeStruct((MEM_DIM, NUM_NODES), jnp.float32),
        name="table_from_linear",
    )(lin)


def _row_spec(shape):
    nd = len(shape)
    return pl.BlockSpec((_RBLK,) + tuple(shape[1:]),
                        lambda i, _nd=nd: (i,) + (0,) * (_nd - 1))


def _full_spec(shape):
    nd = len(shape)
    return pl.BlockSpec(tuple(shape), lambda i, _nd=nd: (0,) * _nd)


def _tc_compute(src_mem, dst_mem, edge_feat, delta_t, weights):
    in_arrays = [src_mem, dst_mem, edge_feat, delta_t] + list(weights)
    in_specs = [_row_spec(src_mem.shape), _row_spec(dst_mem.shape),
                _row_spec(edge_feat.shape), _row_spec(delta_t.shape)]
    in_specs += [_full_spec(w.shape) for w in weights]
    return pl.pallas_call(
        _tc_body,
        grid=(B // _RBLK,),
        in_specs=in_specs,
        out_specs=(_row_spec((B, 128)), _row_spec((B, 1))),
        out_shape=(
            jax.ShapeDtypeStruct((B, 128), jnp.float32),
            jax.ShapeDtypeStruct((B, 1), jnp.float32),
        ),
        name="tgn_dense",
    )(*in_arrays)


def _edge_body(alias_ref, mem_ref, dst_ref, upd_ref, out_ref):
    del alias_ref
    col = dst_ref[...] - NCT * 128
    lanes = lax.broadcasted_iota(jnp.int32, (1, 128), 1)
    p = ((col == lanes) & (col >= 0)).astype(jnp.float32)
    delta = lax.dot_general(p, upd_ref[...], (((0,), (0,)), ((), ())),
                            preferred_element_type=jnp.float32)
    sel = jnp.sum(p, axis=0, keepdims=True) > 0.0
    out_ref[...] = jnp.where(sel, delta.T[:MEM_DIM, :], mem_ref[...])


def _edge_fix(sc_out_t, mem_t, dst_col, upd128):
    return pl.pallas_call(
        _edge_body,
        grid=(1,),
        in_specs=[
            pl.BlockSpec(memory_space=pl.ANY),
            pl.BlockSpec((MEM_DIM, 128), lambda i: (0, NCT)),
            pl.BlockSpec((B, 1), lambda i: (0, 0)),
            pl.BlockSpec((B, 128), lambda i: (0, 0)),
        ],
        out_specs=pl.BlockSpec((MEM_DIM, 128), lambda i: (0, NCT)),
        out_shape=jax.ShapeDtypeStruct((MEM_DIM, NUM_NODES), jnp.float32),
        input_output_aliases={0: 0},
        name="tgn_edge_fix",
    )(sc_out_t, mem_t, dst_col, upd128)


def kernel(src_ids, dst_ids, edge_feat, delta_t, memory,
           gru_w_ih, gru_w_hh, gru_b_ih, gru_b_hh,
           mw1, mb1, mw2, mb2,
           in_proj_w, in_proj_b, out_w, out_b,
           ew1, eb1, ew2, eb2, cw1, cb1, cw2, cb2):
    m = MEM_DIM
    src2d = src_ids.reshape(B // CHUNK, CHUNK).astype(jnp.int32)
    dst2d = dst_ids.reshape(B // CHUNK, CHUNK).astype(jnp.int32)

    mem_t = memory.T
    table_lin = _to_linear(mem_t).reshape(NUM_NODES, MEM_DIM)

    sc_gather, copy_scatter = _get_sc_kernels()
    src_mem, dst_mem = sc_gather(table_lin, src2d, dst2d)
    src_mem = src_mem.reshape(B, m)
    dst_mem = dst_mem.reshape(B, m)

    row = lambda b: b.reshape(1, -1)
    weights = (
        mw1[:, :m].T, mw1[:, m:2 * m].T, mw1[:, 2 * m:2 * m + 2].T,
        row(mw1[:, 2 * m + 2]), row(mb1),
        mw2.T, row(mb2),
        gru_w_ih[:m].T, gru_w_ih[m:2 * m].T, gru_w_ih[2 * m:].T,
        gru_w_hh[:m].T, gru_w_hh[m:2 * m].T, gru_w_hh[2 * m:].T,
        row(gru_b_ih[:m]), row(gru_b_ih[m:2 * m]), row(gru_b_ih[2 * m:]),
        row(gru_b_hh[:m]), row(gru_b_hh[m:2 * m]), row(gru_b_hh[2 * m:]),
        in_proj_w[2 * m:].T, row(in_proj_b[2 * m:]), out_w.T, row(out_b),
        ew1[:, :m].T, ew1[:, m:].T, row(eb1), ew2.T, row(eb2),
        cw1.T, row(cb1), cw2.T, row(cb2),
    )
    upd128, probs2d = _tc_compute(src_mem, dst_mem, edge_feat, delta_t, weights)

    dst_flat = dst_ids.astype(jnp.int32)
    out_t = copy_scatter(mem_t, dst_flat, upd128)
    new_mem_t = _edge_fix(out_t, mem_t, dst_flat.reshape(B, 1), upd128)
    return probs2d.reshape(B), new_mem_t.T


# speedup vs baseline: 1.1708x; 1.1708x over previous
import functools

import jax
import jax.numpy as jnp
from jax import lax
from jax.experimental import pallas as pl
from jax.experimental.pallas import tpu as pltpu
from jax.experimental.pallas import tpu_sc as plsc

NUM_NODES = 1000000
MEM_DIM = 32
B = 16384

NC = 2
NS = 16
NW = NC * NS
CHUNK = 128
ROWS_PER_W = B // NW
NCHUNK = ROWS_PER_W // CHUNK

NCT = NUM_NODES // 128
EDGE_COLS = NUM_NODES - NCT * 128
BASE_CT = NCT // NW
EXTRA_CT = NCT - BASE_CT * NW
MCAP = 768
NIDCH = B // 16
NSCAN = MCAP // 16

@functools.lru_cache(maxsize=None)
def _get_sc_kernels():
    mesh = plsc.VectorSubcoreMesh(core_axis_name="c", subcore_axis_name="s",
                                  num_cores=NC, num_subcores=NS)
    sc_params = pltpu.CompilerParams(use_tc_tiling_on_sc=False)

    @functools.partial(
        pl.kernel,
        mesh=mesh,
        out_type=(
            jax.ShapeDtypeStruct((B // CHUNK, CHUNK, MEM_DIM), jnp.float32),
            jax.ShapeDtypeStruct((B // CHUNK, CHUNK, MEM_DIM), jnp.float32),
        ),
        scratch_types=[
            pltpu.VMEM((NCHUNK, CHUNK), jnp.int32),
            pltpu.VMEM((NCHUNK, CHUNK), jnp.int32),
            pltpu.VMEM((NCHUNK, CHUNK, MEM_DIM), jnp.float32),
            pltpu.VMEM((NCHUNK, CHUNK, MEM_DIM), jnp.float32),
            pltpu.SemaphoreType.DMA,
        ],
        compiler_params=sc_params,
    )
    def sc_gather(table_hbm, src_hbm, dst_hbm, src_out, dst_out,
                  sidx_v, didx_v, srows_v, drows_v, sem):
        wid = lax.axis_index("s") * NC + lax.axis_index("c")
        base = wid * NCHUNK
        pltpu.sync_copy(src_hbm.at[pl.ds(base, NCHUNK)], sidx_v)
        pltpu.sync_copy(dst_hbm.at[pl.ds(base, NCHUNK)], didx_v)
        copies = []
        for c in range(NCHUNK):
            copies.append(
                pltpu.async_copy(table_hbm.at[sidx_v.at[c]], srows_v.at[c], sem))
            copies.append(
                pltpu.async_copy(table_hbm.at[didx_v.at[c]], drows_v.at[c], sem))
        for cp in copies:
            cp.wait()
        pltpu.sync_copy(srows_v, src_out.at[pl.ds(base, NCHUNK)])
        pltpu.sync_copy(drows_v, dst_out.at[pl.ds(base, NCHUNK)])

    cs_params = pltpu.CompilerParams(use_tc_tiling_on_sc=True,
                                     needs_layout_passes=False)
    iota16 = lambda: lax.iota(jnp.int32, 16)

    NRING = 4
    IDCH = 2048

    @functools.partial(
        pl.kernel,
        mesh=mesh,
        out_type=jax.ShapeDtypeStruct((MEM_DIM, NUM_NODES), jnp.float32),
        scratch_types=[
            pltpu.VMEM((IDCH,), jnp.int32),
            pltpu.VMEM((MCAP,), jnp.int32),
            pltpu.VMEM((MCAP,), jnp.int32),
            pltpu.VMEM((MCAP, 128), jnp.float32),
            pltpu.VMEM((NRING, MEM_DIM, 128), jnp.float32),
            pltpu.SemaphoreType.DMA,
            pltpu.SemaphoreType.DMA((NRING,)),
            pltpu.SemaphoreType.DMA((NRING,)),
        ],
        compiler_params=cs_params,
    )
    def copy_scatter(mem_t, dst_hbm, upd_hbm, out_t,
                     ids_v, mid_v, mevt_v, upd_v, slab_v, sem, isem, osem):
        wid = lax.axis_index("s") * NC + lax.axis_index("c")
        extra = jnp.minimum(wid, EXTRA_CT)
        start_ct = wid * BASE_CT + extra
        nct = BASE_CT + jnp.where(wid < EXTRA_CT, 1, 0)
        lo = start_ct * 128
        hi = lo + nct * 128

        neg16 = jnp.full((16,), -1, jnp.int32)
        zero16 = jnp.zeros((16,), jnp.int32)
        for z in range(NSCAN):
            mid_v[pl.ds(z * 16, 16)] = neg16
            mevt_v[pl.ds(z * 16, 16)] = zero16

        def obody(o, cnt):
            pltpu.sync_copy(dst_hbm.at[pl.ds(o * IDCH, IDCH)], ids_v)

            def ebody(i, cnt):
                vec = ids_v[pl.ds(i * 16, 16)]
                mask = (vec >= lo) & (vec < hi)
                mi = mask.astype(jnp.int32)
                pos = cnt + plsc.cumsum(mi) - 1
                plsc.store_scatter(mid_v, [pos], vec, mask=mask)
                plsc.store_scatter(mevt_v, [pos],
                                   (o * IDCH + i * 16) + iota16(), mask=mask)
                return cnt + jnp.sum(mi)

            return lax.fori_loop(0, IDCH // 16, ebody, cnt)

        cnt = lax.fori_loop(0, B // IDCH, obody, jnp.int32(0))

        gcopies = [
            pltpu.async_copy(upd_hbm.at[mevt_v.at[pl.ds(j * 128, 128)]],
                             upd_v.at[pl.ds(j * 128, 128)], sem)
            for j in range(MCAP // 128)
        ]
        for cp in gcopies:
            cp.wait()

        def start_in(b, ct):
            c0 = pl.multiple_of((start_ct + ct) * 128, 128)
            pltpu.async_copy(mem_t.at[:, pl.ds(c0, 128)], slab_v.at[b],
                             isem.at[b])

        def wait_in(b):
            pltpu.make_async_copy(mem_t.at[:, pl.ds(0, 128)], slab_v.at[b],
                                  isem.at[b]).wait()

        def start_out(b, ct):
            c0 = pl.multiple_of((start_ct + ct) * 128, 128)
            pltpu.async_copy(slab_v.at[b], out_t.at[:, pl.ds(c0, 128)],
                             osem.at[b])

        def wait_out(b):
            pltpu.make_async_copy(slab_v.at[b], out_t.at[:, pl.ds(0, 128)],
                                  osem.at[b]).wait()

        def process(b, ct):
            c0 = (start_ct + ct) * 128

            def sbody(j, _s):
                evec = jnp.full((16,), j * 16, jnp.int32) + iota16()
                vec = mid_v[pl.ds(j * 16, 16)]
                msk = (vec >= c0) & (vec < c0 + 128) & (evec < cnt)
                nhit = jnp.sum(msk.astype(jnp.int32))

                @pl.when(nhit > 0)
                def _():
                    col = vec - c0
                    for d in range(MEM_DIM):
                        dfull = jnp.full((16,), d, jnp.int32)
                        vals = plsc.load_gather(upd_v, [evec, dfull])
                        plsc.store_scatter(slab_v.at[b], [dfull, col],
                                           vals, mask=msk)
                return 0

            lax.fori_loop(0, NSCAN, sbody, 0)

        for b in range(NRING):
            start_in(b, b)

        def gbody(g, _):
            for b in range(NRING):
                ct = g * NRING + b
                wait_in(b)
                process(b, ct)
                start_out(b, ct)
                bprev = (b - 1) % NRING
                ctprev = ct - 1

                def _retire():
                    wait_out(bprev)

                    @pl.when(ctprev + NRING < BASE_CT)
                    def _():
                        start_in(bprev, ctprev + NRING)

                if b > 0:
                    _retire()
                else:
                    pl.when(g > 0)(_retire)
            return 0

        lax.fori_loop(0, BASE_CT // NRING, gbody, 0)
        wait_out(NRING - 1)

        @pl.when(wid < EXTRA_CT)
        def _():
            start_in(0, BASE_CT)
            wait_in(0)
            process(0, BASE_CT)
            start_out(0, BASE_CT)
            wait_out(0)

    return sc_gather, copy_scatter


_RBLK = 2048


def _tc_body(src_ref, dst_ref, edge_ref, dt_ref,
             w1s_ref, w1d_ref, w1e_ref, w1t_ref, b1_ref,
             w2_ref, b2_ref,
             wih_r_ref, wih_z_ref, wih_n_ref,
             whh_r_ref, whh_z_ref, whh_n_ref,
             bi_r_ref, bi_z_ref, bi_n_ref,
             bh_r_ref, bh_z_ref, bh_n_ref,
             wv_ref, bv_ref, wout_ref, bout_ref,
             we1a_ref, we1e_ref, be1_ref, we2_ref, be2_ref,
             wc1_ref, bc1_ref, wc2_ref, bc2_ref,
             upd_ref, probs_ref):
    src = src_ref[...]
    dst = dst_ref[...]
    edge = edge_ref[...]
    dt = dt_ref[...]

    def mm(a, w):
        return jnp.dot(a, w[...], preferred_element_type=jnp.float32)

    h = mm(src, w1s_ref) + mm(dst, w1d_ref) + mm(edge, w1e_ref) \
        + dt * w1t_ref[...] + b1_ref[...]
    h = jnp.maximum(h, 0.0)
    msg = mm(h, w2_ref) + b2_ref[...]

    r = jax.nn.sigmoid(mm(msg, wih_r_ref) + bi_r_ref[...]
                       + mm(dst, whh_r_ref) + bh_r_ref[...])
    z = jax.nn.sigmoid(mm(msg, wih_z_ref) + bi_z_ref[...]
                       + mm(dst, whh_z_ref) + bh_z_ref[...])
    n = jnp.tanh(mm(msg, wih_n_ref) + bi_n_ref[...]
                 + r * (mm(dst, whh_n_ref) + bh_n_ref[...]))
    updated = (1.0 - z) * n + z * dst
    upd_ref[...] = jnp.concatenate(
        [updated, jnp.zeros((updated.shape[0], 128 - MEM_DIM), jnp.float32)],
        axis=1)

    v = mm(dst, wv_ref) + bv_ref[...]
    attn_out = mm(v, wout_ref) + bout_ref[...]
    e = jnp.maximum(mm(attn_out, we1a_ref) + mm(edge, we1e_ref) + be1_ref[...], 0.0)
    e = mm(e, we2_ref) + be2_ref[...]

    c = jnp.maximum(mm(e, wc1_ref) + bc1_ref[...], 0.0)
    logits = mm(c, wc2_ref) + bc2_ref[...]
    probs_ref[...] = jax.nn.sigmoid(logits)


_TW = 2048
_TR = _TW * MEM_DIM // 128
_TGRID = -(-NUM_NODES // _TW)
_PACKED_ROWS = NUM_NODES * MEM_DIM // 128


def _to_linear_body(mem_t_ref, out_ref):
    t1 = mem_t_ref[...].T
    t3 = t1.reshape(_TR, 4, MEM_DIM)
    out_ref[...] = jnp.concatenate([t3[:, a, :] for a in range(4)], axis=1)


def _from_linear_body(lin_ref, out_ref):
    blk = lin_ref[...]
    parts = [blk[:, MEM_DIM * a:MEM_DIM * (a + 1)] for a in range(4)]
    st = jnp.stack(parts, axis=1)
    out_ref[...] = st.reshape(_TW, MEM_DIM).T


def _to_linear(mem_t):
    return pl.pallas_call(
        _to_linear_body,
        grid=(_TGRID,),
        in_specs=[pl.BlockSpec((MEM_DIM, _TW), lambda i: (0, i))],
        out_specs=pl.BlockSpec((_TR, 128), lambda i: (i, 0)),
        out_shape=jax.ShapeDtypeStruct((_PACKED_ROWS, 128), jnp.float32),
        name="table_to_linear",
    )(mem_t)


def _from_linear(lin):
    return pl.pallas_call(
        _from_linear_body,
        grid=(_TGRID,),
        in_specs=[pl.BlockSpec((_TR, 128), lambda i: (i, 0))],
        out_specs=pl.BlockSpec((MEM_DIM, _TW), lambda i: (0, i)),
        out_shape=jax.ShapeDtypeStruct((MEM_DIM, NUM_NODES), jnp.float32),
        name="table_from_linear",
    )(lin)


def _row_spec(shape):
    nd = len(shape)
    return pl.BlockSpec((_RBLK,) + tuple(shape[1:]),
                        lambda i, _nd=nd: (i,) + (0,) * (_nd - 1))


def _full_spec(shape):
    nd = len(shape)
    return pl.BlockSpec(tuple(shape), lambda i, _nd=nd: (0,) * _nd)


def _tc_compute(src_mem, dst_mem, edge_feat, delta_t, weights):
    in_arrays = [src_mem, dst_mem, edge_feat, delta_t] + list(weights)
    in_specs = [_row_spec(src_mem.shape), _row_spec(dst_mem.shape),
                _row_spec(edge_feat.shape), _row_spec(delta_t.shape)]
    in_specs += [_full_spec(w.shape) for w in weights]
    return pl.pallas_call(
        _tc_body,
        grid=(B // _RBLK,),
        in_specs=in_specs,
        out_specs=(_row_spec((B, 128)), _row_spec((B, 1))),
        out_shape=(
            jax.ShapeDtypeStruct((B, 128), jnp.float32),
            jax.ShapeDtypeStruct((B, 1), jnp.float32),
        ),
        name="tgn_dense",
    )(*in_arrays)


def _edge_body(alias_ref, mem_ref, dst_ref, upd_ref, out_ref):
    del alias_ref
    col = dst_ref[...] - NCT * 128
    lanes = lax.broadcasted_iota(jnp.int32, (1, 128), 1)
    p = ((col == lanes) & (col >= 0)).astype(jnp.float32)
    delta = lax.dot_general(p, upd_ref[...], (((0,), (0,)), ((), ())),
                            preferred_element_type=jnp.float32)
    sel = jnp.sum(p, axis=0, keepdims=True) > 0.0
    out_ref[...] = jnp.where(sel, delta.T[:MEM_DIM, :], mem_ref[...])


def _edge_fix(sc_out_t, mem_t, dst_col, upd128):
    return pl.pallas_call(
        _edge_body,
        grid=(1,),
        in_specs=[
            pl.BlockSpec(memory_space=pl.ANY),
            pl.BlockSpec((MEM_DIM, 128), lambda i: (0, NCT)),
            pl.BlockSpec((B, 1), lambda i: (0, 0)),
            pl.BlockSpec((B, 128), lambda i: (0, 0)),
        ],
        out_specs=pl.BlockSpec((MEM_DIM, 128), lambda i: (0, NCT)),
        out_shape=jax.ShapeDtypeStruct((MEM_DIM, NUM_NODES), jnp.float32),
        input_output_aliases={0: 0},
        name="tgn_edge_fix",
    )(sc_out_t, mem_t, dst_col, upd128)


def kernel(src_ids, dst_ids, edge_feat, delta_t, memory,
           gru_w_ih, gru_w_hh, gru_b_ih, gru_b_hh,
           mw1, mb1, mw2, mb2,
           in_proj_w, in_proj_b, out_w, out_b,
           ew1, eb1, ew2, eb2, cw1, cb1, cw2, cb2):
    m = MEM_DIM
    src2d = src_ids.reshape(B // CHUNK, CHUNK).astype(jnp.int32)
    dst2d = dst_ids.reshape(B // CHUNK, CHUNK).astype(jnp.int32)

    mem_t = memory.T
    table_lin = _to_linear(mem_t).reshape(NUM_NODES, MEM_DIM)

    sc_gather, copy_scatter = _get_sc_kernels()
    src_mem, dst_mem = sc_gather(table_lin, src2d, dst2d)
    src_mem = src_mem.reshape(B, m)
    dst_mem = dst_mem.reshape(B, m)

    row = lambda b: b.reshape(1, -1)
    weights = (
        mw1[:, :m].T, mw1[:, m:2 * m].T, mw1[:, 2 * m:2 * m + 2].T,
        row(mw1[:, 2 * m + 2]), row(mb1),
        mw2.T, row(mb2),
        gru_w_ih[:m].T, gru_w_ih[m:2 * m].T, gru_w_ih[2 * m:].T,
        gru_w_hh[:m].T, gru_w_hh[m:2 * m].T, gru_w_hh[2 * m:].T,
        row(gru_b_ih[:m]), row(gru_b_ih[m:2 * m]), row(gru_b_ih[2 * m:]),
        row(gru_b_hh[:m]), row(gru_b_hh[m:2 * m]), row(gru_b_hh[2 * m:]),
        in_proj_w[2 * m:].T, row(in_proj_b[2 * m:]), out_w.T, row(out_b),
        ew1[:, :m].T, ew1[:, m:].T, row(eb1), ew2.T, row(eb2),
        cw1.T, row(cb1), cw2.T, row(cb2),
    )
    upd128, probs2d = _tc_compute(src_mem, dst_mem, edge_feat, delta_t, weights)

    dst_flat = dst_ids.astype(jnp.int32)
    out_t = copy_scatter(mem_t, dst_flat, upd128)
    new_mem_t = _edge_fix(out_t, mem_t, dst_flat.reshape(B, 1), upd128)
    return probs2d.reshape(B), new_mem_t.T


# speedup vs baseline: 1.5511x; 1.3248x over previous
import functools

import jax
import jax.numpy as jnp
from jax import lax
from jax.experimental import pallas as pl
from jax.experimental.pallas import tpu as pltpu
from jax.experimental.pallas import tpu_sc as plsc

NUM_NODES = 1000000
MEM_DIM = 32
B = 16384

NC = 2
NS = 16
NW = NC * NS
CHUNK = 128
ROWS_PER_W = B // NW
NCHUNK = ROWS_PER_W // CHUNK

@functools.lru_cache(maxsize=None)
def _get_sc_kernels():
    mesh = plsc.VectorSubcoreMesh(core_axis_name="c", subcore_axis_name="s",
                                  num_cores=NC, num_subcores=NS)
    sc_params = pltpu.CompilerParams(use_tc_tiling_on_sc=False)

    @functools.partial(
        pl.kernel,
        mesh=mesh,
        out_type=(
            jax.ShapeDtypeStruct((B // CHUNK, CHUNK, MEM_DIM), jnp.float32),
            jax.ShapeDtypeStruct((B // CHUNK, CHUNK, MEM_DIM), jnp.float32),
        ),
        scratch_types=[
            pltpu.VMEM((NCHUNK, CHUNK), jnp.int32),
            pltpu.VMEM((NCHUNK, CHUNK), jnp.int32),
            pltpu.VMEM((NCHUNK, CHUNK, MEM_DIM), jnp.float32),
            pltpu.VMEM((NCHUNK, CHUNK, MEM_DIM), jnp.float32),
            pltpu.SemaphoreType.DMA,
        ],
        compiler_params=sc_params,
    )
    def sc_gather(table_hbm, src_hbm, dst_hbm, src_out, dst_out,
                  sidx_v, didx_v, srows_v, drows_v, sem):
        wid = lax.axis_index("s") * NC + lax.axis_index("c")
        base = wid * NCHUNK
        pltpu.sync_copy(src_hbm.at[pl.ds(base, NCHUNK)], sidx_v)
        pltpu.sync_copy(dst_hbm.at[pl.ds(base, NCHUNK)], didx_v)
        copies = []
        for c in range(NCHUNK):
            copies.append(
                pltpu.async_copy(table_hbm.at[sidx_v.at[c]], srows_v.at[c], sem))
            copies.append(
                pltpu.async_copy(table_hbm.at[didx_v.at[c]], drows_v.at[c], sem))
        for cp in copies:
            cp.wait()
        pltpu.sync_copy(srows_v, src_out.at[pl.ds(base, NCHUNK)])
        pltpu.sync_copy(drows_v, dst_out.at[pl.ds(base, NCHUNK)])

    @functools.partial(
        pl.kernel,
        mesh=mesh,
        out_type=(),
        scratch_types=[
            pltpu.VMEM((NCHUNK, CHUNK), jnp.int32),
            pltpu.VMEM((NCHUNK, CHUNK, MEM_DIM), jnp.float32),
            pltpu.SemaphoreType.DMA,
        ],
        compiler_params=sc_params,
    )
    def sc_scatter(dst_hbm, upd_hbm, table_ref, didx_v, rows_v, sem):
        wid = lax.axis_index("s") * NC + lax.axis_index("c")
        base = wid * NCHUNK
        pltpu.sync_copy(dst_hbm.at[pl.ds(base, NCHUNK)], didx_v)
        pltpu.sync_copy(upd_hbm.at[pl.ds(base, NCHUNK)], rows_v)
        copies = []
        for c in range(NCHUNK):
            copies.append(
                pltpu.async_copy(rows_v.at[c], table_ref.at[didx_v.at[c]], sem))
        for cp in copies:
            cp.wait()

    return sc_gather, sc_scatter


_RBLK = 2048


def _tc_body(src_ref, dst_ref, edge_ref, dt_ref,
             w1s_ref, w1d_ref, w1e_ref, w1t_ref, b1_ref,
             w2_ref, b2_ref,
             wih_r_ref, wih_z_ref, wih_n_ref,
             whh_r_ref, whh_z_ref, whh_n_ref,
             bi_r_ref, bi_z_ref, bi_n_ref,
             bh_r_ref, bh_z_ref, bh_n_ref,
             wv_ref, bv_ref, wout_ref, bout_ref,
             we1a_ref, we1e_ref, be1_ref, we2_ref, be2_ref,
             wc1_ref, bc1_ref, wc2_ref, bc2_ref,
             upd_ref, probs_ref):
    src = src_ref[...]
    dst = dst_ref[...]
    edge = edge_ref[...]
    dt = dt_ref[...]

    def mm(a, w):
        return jnp.dot(a, w[...], preferred_element_type=jnp.float32)

    h = mm(src, w1s_ref) + mm(dst, w1d_ref) + mm(edge, w1e_ref) \
        + dt * w1t_ref[...] + b1_ref[...]
    h = jnp.maximum(h, 0.0)
    msg = mm(h, w2_ref) + b2_ref[...]

    r = jax.nn.sigmoid(mm(msg, wih_r_ref) + bi_r_ref[...]
                       + mm(dst, whh_r_ref) + bh_r_ref[...])
    z = jax.nn.sigmoid(mm(msg, wih_z_ref) + bi_z_ref[...]
                       + mm(dst, whh_z_ref) + bh_z_ref[...])
    n = jnp.tanh(mm(msg, wih_n_ref) + bi_n_ref[...]
                 + r * (mm(dst, whh_n_ref) + bh_n_ref[...]))
    upd_ref[...] = (1.0 - z) * n + z * dst

    v = mm(dst, wv_ref) + bv_ref[...]
    attn_out = mm(v, wout_ref) + bout_ref[...]
    e = jnp.maximum(mm(attn_out, we1a_ref) + mm(edge, we1e_ref) + be1_ref[...], 0.0)
    e = mm(e, we2_ref) + be2_ref[...]

    c = jnp.maximum(mm(e, wc1_ref) + bc1_ref[...], 0.0)
    logits = mm(c, wc2_ref) + bc2_ref[...]
    probs_ref[...] = jax.nn.sigmoid(logits)


_TW = 2048
_TR = _TW * MEM_DIM // 128
_TGRID = -(-NUM_NODES // _TW)
_PACKED_ROWS = NUM_NODES * MEM_DIM // 128


def _to_linear_body(mem_t_ref, out_ref):
    t1 = mem_t_ref[...].T
    t3 = t1.reshape(_TR, 4, MEM_DIM)
    out_ref[...] = jnp.concatenate([t3[:, a, :] for a in range(4)], axis=1)


def _from_linear_body(lin_ref, out_ref):
    blk = lin_ref[...]
    parts = [blk[:, MEM_DIM * a:MEM_DIM * (a + 1)] for a in range(4)]
    st = jnp.stack(parts, axis=1)
    out_ref[...] = st.reshape(_TW, MEM_DIM).T


def _to_linear(mem_t):
    return pl.pallas_call(
        _to_linear_body,
        grid=(_TGRID,),
        in_specs=[pl.BlockSpec((MEM_DIM, _TW), lambda i: (0, i))],
        out_specs=pl.BlockSpec((_TR, 128), lambda i: (i, 0)),
        out_shape=jax.ShapeDtypeStruct((_PACKED_ROWS, 128), jnp.float32),
        name="table_to_linear",
    )(mem_t)


def _from_linear(lin):
    return pl.pallas_call(
        _from_linear_body,
        grid=(_TGRID,),
        in_specs=[pl.BlockSpec((_TR, 128), lambda i: (i, 0))],
        out_specs=pl.BlockSpec((MEM_DIM, _TW), lambda i: (0, i)),
        out_shape=jax.ShapeDtypeStruct((MEM_DIM, NUM_NODES), jnp.float32),
        name="table_from_linear",
    )(lin)


def _row_spec(shape):
    nd = len(shape)
    return pl.BlockSpec((_RBLK,) + tuple(shape[1:]),
                        lambda i, _nd=nd: (i,) + (0,) * (_nd - 1))


def _full_spec(shape):
    nd = len(shape)
    return pl.BlockSpec(tuple(shape), lambda i, _nd=nd: (0,) * _nd)


def _tc_compute(src_mem, dst_mem, edge_feat, delta_t, weights):
    in_arrays = [src_mem, dst_mem, edge_feat, delta_t] + list(weights)
    in_specs = [_row_spec(src_mem.shape), _row_spec(dst_mem.shape),
                _row_spec(edge_feat.shape), _row_spec(delta_t.shape)]
    in_specs += [_full_spec(w.shape) for w in weights]
    return pl.pallas_call(
        _tc_body,
        grid=(B // _RBLK,),
        in_specs=in_specs,
        out_specs=(_row_spec((B, MEM_DIM)), _row_spec((B, 1))),
        out_shape=(
            jax.ShapeDtypeStruct((B, MEM_DIM), jnp.float32),
            jax.ShapeDtypeStruct((B, 1), jnp.float32),
        ),
        name="tgn_dense",
    )(*in_arrays)


def kernel(src_ids, dst_ids, edge_feat, delta_t, memory,
           gru_w_ih, gru_w_hh, gru_b_ih, gru_b_hh,
           mw1, mb1, mw2, mb2,
           in_proj_w, in_proj_b, out_w, out_b,
           ew1, eb1, ew2, eb2, cw1, cb1, cw2, cb2):
    m = MEM_DIM
    src2d = src_ids.reshape(B // CHUNK, CHUNK).astype(jnp.int32)
    dst2d = dst_ids.reshape(B // CHUNK, CHUNK).astype(jnp.int32)

    table_lin = _to_linear(memory.T).reshape(NUM_NODES, MEM_DIM)

    sc_gather, sc_scatter = _get_sc_kernels()
    src_mem, dst_mem = sc_gather(table_lin, src2d, dst2d)
    src_mem = src_mem.reshape(B, m)
    dst_mem = dst_mem.reshape(B, m)

    row = lambda b: b.reshape(1, -1)
    weights = (
        mw1[:, :m].T, mw1[:, m:2 * m].T, mw1[:, 2 * m:2 * m + 2].T,
        row(mw1[:, 2 * m + 2]), row(mb1),
        mw2.T, row(mb2),
        gru_w_ih[:m].T, gru_w_ih[m:2 * m].T, gru_w_ih[2 * m:].T,
        gru_w_hh[:m].T, gru_w_hh[m:2 * m].T, gru_w_hh[2 * m:].T,
        row(gru_b_ih[:m]), row(gru_b_ih[m:2 * m]), row(gru_b_ih[2 * m:]),
        row(gru_b_hh[:m]), row(gru_b_hh[m:2 * m]), row(gru_b_hh[2 * m:]),
        in_proj_w[2 * m:].T, row(in_proj_b[2 * m:]), out_w.T, row(out_b),
        ew1[:, :m].T, ew1[:, m:].T, row(eb1), ew2.T, row(eb2),
        cw1.T, row(cb1), cw2.T, row(cb2),
    )
    updated, probs2d = _tc_compute(src_mem, dst_mem, edge_feat, delta_t, weights)

    table_ref = jax.new_ref(table_lin)
    sc_scatter(dst2d, updated.reshape(B // CHUNK, CHUNK, m), table_ref)
    return probs2d.reshape(B), table_ref[...]


# speedup vs baseline: 1.5518x; 1.0005x over previous
import functools

import jax
import jax.numpy as jnp
from jax import lax
from jax.experimental import pallas as pl
from jax.experimental.pallas import tpu as pltpu
from jax.experimental.pallas import tpu_sc as plsc

NUM_NODES = 1000000
MEM_DIM = 32
B = 16384

NC = 2
NS = 16
NW = NC * NS
CHUNK = 128
ROWS_PER_W = B // NW
NCHUNK = ROWS_PER_W // CHUNK

@functools.lru_cache(maxsize=None)
def _get_sc_kernels():
    mesh = plsc.VectorSubcoreMesh(core_axis_name="c", subcore_axis_name="s",
                                  num_cores=NC, num_subcores=NS)
    sc_params = pltpu.CompilerParams(use_tc_tiling_on_sc=False)

    @functools.partial(
        pl.kernel,
        mesh=mesh,
        out_type=(
            jax.ShapeDtypeStruct((B // CHUNK, CHUNK, MEM_DIM), jnp.float32),
            jax.ShapeDtypeStruct((B // CHUNK, CHUNK, MEM_DIM), jnp.float32),
        ),
        scratch_types=[
            pltpu.VMEM((NCHUNK, CHUNK), jnp.int32),
            pltpu.VMEM((NCHUNK, CHUNK), jnp.int32),
            pltpu.VMEM((NCHUNK, CHUNK, MEM_DIM), jnp.float32),
            pltpu.VMEM((NCHUNK, CHUNK, MEM_DIM), jnp.float32),
            pltpu.SemaphoreType.DMA,
        ],
        compiler_params=sc_params,
    )
    def sc_gather(table_hbm, src_hbm, dst_hbm, src_out, dst_out,
                  sidx_v, didx_v, srows_v, drows_v, sem):
        wid = lax.axis_index("s") * NC + lax.axis_index("c")
        base = wid * NCHUNK
        pltpu.sync_copy(src_hbm.at[pl.ds(base, NCHUNK)], sidx_v)
        pltpu.sync_copy(dst_hbm.at[pl.ds(base, NCHUNK)], didx_v)
        copies = []
        for c in range(NCHUNK):
            copies.append(
                pltpu.async_copy(table_hbm.at[sidx_v.at[c]], srows_v.at[c], sem))
            copies.append(
                pltpu.async_copy(table_hbm.at[didx_v.at[c]], drows_v.at[c], sem))
        for cp in copies:
            cp.wait()
        pltpu.sync_copy(srows_v, src_out.at[pl.ds(base, NCHUNK)])
        pltpu.sync_copy(drows_v, dst_out.at[pl.ds(base, NCHUNK)])

    @functools.partial(
        pl.kernel,
        mesh=mesh,
        out_type=(),
        scratch_types=[
            pltpu.VMEM((NCHUNK, CHUNK), jnp.int32),
            pltpu.VMEM((NCHUNK, CHUNK, MEM_DIM), jnp.float32),
            pltpu.SemaphoreType.DMA,
        ],
        compiler_params=sc_params,
    )
    def sc_scatter(dst_hbm, upd_hbm, table_ref, didx_v, rows_v, sem):
        wid = lax.axis_index("s") * NC + lax.axis_index("c")
        base = wid * NCHUNK
        pltpu.sync_copy(dst_hbm.at[pl.ds(base, NCHUNK)], didx_v)
        pltpu.sync_copy(upd_hbm.at[pl.ds(base, NCHUNK)], rows_v)
        copies = []
        for c in range(NCHUNK):
            copies.append(
                pltpu.async_copy(rows_v.at[c], table_ref.at[didx_v.at[c]], sem))
        for cp in copies:
            cp.wait()

    return sc_gather, sc_scatter


_RBLK = 2048


def _tc_body(src_ref, dst_ref, edge_ref, dt_ref,
             w1s_ref, w1d_ref, w1e_ref, w1t_ref, b1_ref,
             w2_ref, b2_ref,
             wih_r_ref, wih_z_ref, wih_n_ref,
             whh_r_ref, whh_z_ref, whh_n_ref,
             bi_r_ref, bi_z_ref, bi_n_ref,
             bh_r_ref, bh_z_ref, bh_n_ref,
             wv_ref, bv_ref, wout_ref, bout_ref,
             we1a_ref, we1e_ref, be1_ref, we2_ref, be2_ref,
             wc1_ref, bc1_ref, wc2_ref, bc2_ref,
             upd_ref, probs_ref):
    src = src_ref[...]
    dst = dst_ref[...]
    edge = edge_ref[...]
    dt = dt_ref[...]

    def mm(a, w):
        return jnp.dot(a, w[...], preferred_element_type=jnp.float32)

    h = mm(src, w1s_ref) + mm(dst, w1d_ref) + mm(edge, w1e_ref) \
        + dt * w1t_ref[...] + b1_ref[...]
    h = jnp.maximum(h, 0.0)
    msg = mm(h, w2_ref) + b2_ref[...]

    r = jax.nn.sigmoid(mm(msg, wih_r_ref) + bi_r_ref[...]
                       + mm(dst, whh_r_ref) + bh_r_ref[...])
    z = jax.nn.sigmoid(mm(msg, wih_z_ref) + bi_z_ref[...]
                       + mm(dst, whh_z_ref) + bh_z_ref[...])
    n = jnp.tanh(mm(msg, wih_n_ref) + bi_n_ref[...]
                 + r * (mm(dst, whh_n_ref) + bh_n_ref[...]))
    upd_ref[...] = (1.0 - z) * n + z * dst

    v = mm(dst, wv_ref) + bv_ref[...]
    attn_out = mm(v, wout_ref) + bout_ref[...]
    e = jnp.maximum(mm(attn_out, we1a_ref) + mm(edge, we1e_ref) + be1_ref[...], 0.0)
    e = mm(e, we2_ref) + be2_ref[...]

    c = jnp.maximum(mm(e, wc1_ref) + bc1_ref[...], 0.0)
    logits = mm(c, wc2_ref) + bc2_ref[...]
    probs_ref[...] = jax.nn.sigmoid(logits)


_TW = 2048
_TR = _TW * MEM_DIM // 128
_TGRID = -(-NUM_NODES // _TW)
_PACKED_ROWS = NUM_NODES * MEM_DIM // 128


def _to_linear_body(mem_t_ref, out_ref):
    t1 = mem_t_ref[...].T
    t3 = t1.reshape(_TR, 4, MEM_DIM)
    out_ref[...] = jnp.concatenate([t3[:, a, :] for a in range(4)], axis=1)


def _to_linear(mem_t):
    return pl.pallas_call(
        _to_linear_body,
        grid=(_TGRID,),
        in_specs=[pl.BlockSpec((MEM_DIM, _TW), lambda i: (0, i))],
        out_specs=pl.BlockSpec((_TR, 128), lambda i: (i, 0)),
        out_shape=jax.ShapeDtypeStruct((_PACKED_ROWS, 128), jnp.float32),
        name="table_to_linear",
    )(mem_t)


def _row_spec(shape):
    nd = len(shape)
    return pl.BlockSpec((_RBLK,) + tuple(shape[1:]),
                        lambda i, _nd=nd: (i,) + (0,) * (_nd - 1))


def _full_spec(shape):
    nd = len(shape)
    return pl.BlockSpec(tuple(shape), lambda i, _nd=nd: (0,) * _nd)


def _tc_compute(src_mem, dst_mem, edge_feat, delta_t, weights):
    in_arrays = [src_mem, dst_mem, edge_feat, delta_t] + list(weights)
    in_specs = [_row_spec(src_mem.shape), _row_spec(dst_mem.shape),
                _row_spec(edge_feat.shape), _row_spec(delta_t.shape)]
    in_specs += [_full_spec(w.shape) for w in weights]
    return pl.pallas_call(
        _tc_body,
        grid=(B // _RBLK,),
        in_specs=in_specs,
        out_specs=(_row_spec((B, MEM_DIM)), _row_spec((B, 1))),
        out_shape=(
            jax.ShapeDtypeStruct((B, MEM_DIM), jnp.float32),
            jax.ShapeDtypeStruct((B, 1), jnp.float32),
        ),
        name="tgn_dense",
    )(*in_arrays)


def kernel(src_ids, dst_ids, edge_feat, delta_t, memory,
           gru_w_ih, gru_w_hh, gru_b_ih, gru_b_hh,
           mw1, mb1, mw2, mb2,
           in_proj_w, in_proj_b, out_w, out_b,
           ew1, eb1, ew2, eb2, cw1, cb1, cw2, cb2):
    m = MEM_DIM
    src2d = src_ids.reshape(B // CHUNK, CHUNK).astype(jnp.int32)
    dst2d = dst_ids.reshape(B // CHUNK, CHUNK).astype(jnp.int32)

    table_lin = _to_linear(memory.T).reshape(NUM_NODES, MEM_DIM)

    sc_gather, sc_scatter = _get_sc_kernels()
    src_mem, dst_mem = sc_gather(table_lin, src2d, dst2d)
    src_mem = src_mem.reshape(B, m)
    dst_mem = dst_mem.reshape(B, m)

    row = lambda b: b.reshape(1, -1)
    weights = (
        mw1[:, :m].T, mw1[:, m:2 * m].T, mw1[:, 2 * m:2 * m + 2].T,
        row(mw1[:, 2 * m + 2]), row(mb1),
        mw2.T, row(mb2),
        gru_w_ih[:m].T, gru_w_ih[m:2 * m].T, gru_w_ih[2 * m:].T,
        gru_w_hh[:m].T, gru_w_hh[m:2 * m].T, gru_w_hh[2 * m:].T,
        row(gru_b_ih[:m]), row(gru_b_ih[m:2 * m]), row(gru_b_ih[2 * m:]),
        row(gru_b_hh[:m]), row(gru_b_hh[m:2 * m]), row(gru_b_hh[2 * m:]),
        in_proj_w[2 * m:].T, row(in_proj_b[2 * m:]), out_w.T, row(out_b),
        ew1[:, :m].T, ew1[:, m:].T, row(eb1), ew2.T, row(eb2),
        cw1.T, row(cb1), cw2.T, row(cb2),
    )
    updated, probs2d = _tc_compute(src_mem, dst_mem, edge_feat, delta_t, weights)

    table_ref = jax.new_ref(table_lin)
    sc_scatter(dst2d, updated.reshape(B // CHUNK, CHUNK, m), table_ref)
    return probs2d.reshape(B), table_ref[...]
